# baseline (speedup 1.0000x reference)
import functools

import jax
import jax.numpy as jnp
from jax import lax
from jax.experimental import pallas as pl
from jax.experimental.pallas import tpu as pltpu
from jax.experimental.pallas import tpu_sc as plsc

_M = 48
_K = 256
_DSUB = 16
_B = 16384
_D = _M * _DSUB

_NC = 2
_NS = 16
_NW = _NC * _NS
_TOTAL = _B * _M
_PER_W = _TOTAL // _NW
_RPG = 128
_NG = _PER_W // _RPG
_KF = 8
_BURST = _KF * _RPG
_NB = _NG // _KF

_mesh = plsc.VectorSubcoreMesh(core_axis_name="c", subcore_axis_name="s")


@functools.partial(
    pl.kernel,
    mesh=_mesh,
    out_type=jax.ShapeDtypeStruct((_TOTAL, _DSUB), jnp.float32),
    scratch_types=[
        pltpu.VMEM((_NG, _RPG), jnp.int32),
        pltpu.VMEM((3, _BURST, _DSUB), jnp.float32),
        pltpu.SemaphoreType.DMA,
        pltpu.SemaphoreType.DMA,
        pltpu.SemaphoreType.DMA,
    ],
    compiler_params=pltpu.CompilerParams(use_tc_tiling_on_sc=False),
)
def _pq_gather(codes_hbm, table_hbm, out_hbm, idx_v, rows_v, sem_ga, sem_gb,
               sem_s):
    wid = lax.axis_index("s") * _NC + lax.axis_index("c")

    pltpu.sync_copy(codes_hbm.at[pl.ds(wid * _NG, _NG)], idx_v)

    lane = lax.iota(jnp.int32, 16)

    def add_offsets(j, carry):
        for o in range(_RPG // 16):
            pos = j * _RPG + (o * 16) + lane
            off = lax.rem(pos, _M) * _K
            sl = pl.ds(o * 16, 16)
            idx_v[j, sl] = idx_v[j, sl] + off
        return carry

    lax.fori_loop(0, _NG, add_offsets, 0)

    def fire(g, buf, sem):
        for f in range(_KF):
            pltpu.async_copy(
                table_hbm.at[idx_v.at[g * _KF + f]],
                rows_v.at[buf, pl.ds(f * _RPG, _RPG)],
                sem,
            )

    def drain_gathers(buf, sem):
        pltpu.make_async_copy(
            out_hbm.at[pl.ds(0, _BURST)], rows_v.at[buf], sem
        ).wait()

    def scatter(g, buf):
        pltpu.async_copy(
            rows_v.at[buf],
            out_hbm.at[pl.ds(wid * _PER_W + g * _BURST, _BURST)],
            sem_s,
        )

    def drain_scatter(buf):
        pltpu.make_async_copy(
            rows_v.at[buf], out_hbm.at[pl.ds(wid * _PER_W, _BURST)], sem_s
        ).wait()

    sems = (sem_ga, sem_gb)

    def burst_pair(i, carry):
        for b2 in range(2):
            g = 2 * i + b2
            buf = lax.rem(g, 3)

            @pl.when(g >= 3)
            def _scatter_done():
                drain_scatter(buf)

            fire(g, buf, sems[b2])

            @pl.when(g >= 1)
            def _prev_done():
                prev = lax.rem(g + 2, 3)
                drain_gathers(prev, sems[1 - b2])
                scatter(g - 1, prev)

        return carry

    lax.fori_loop(0, _NB // 2, burst_pair, 0)

    last = lax.rem(_NB - 1, 3)
    drain_gathers(last, sems[(_NB - 1) % 2])
    scatter(_NB - 1, last)
    for _ in range(3):
        drain_scatter(0)


def kernel(doc_codes, sub_weights):
    codes = doc_codes.astype(jnp.int32).reshape(_NW * _NG, _RPG)
    table = sub_weights.reshape(_M * _K, _DSUB)
    flat = _pq_gather(codes, table)
    return flat.reshape(_B, _D)


# speedup vs baseline: 1.0040x; 1.0040x over previous
import functools

import jax
import jax.numpy as jnp
from jax import lax
from jax.experimental import pallas as pl
from jax.experimental.pallas import tpu as pltpu
from jax.experimental.pallas import tpu_sc as plsc

_M = 48
_K = 256
_DSUB = 16
_B = 16384
_D = _M * _DSUB

_NC = 2
_NS = 16
_NW = _NC * _NS
_TOTAL = _B * _M
_PER_W = _TOTAL // _NW
_RPG = 128
_NG = _PER_W // _RPG
_KF = 8
_BURST = _KF * _RPG
_NB = _NG // _KF

_mesh = plsc.VectorSubcoreMesh(core_axis_name="c", subcore_axis_name="s")


@functools.partial(
    pl.kernel,
    mesh=_mesh,
    out_type=jax.ShapeDtypeStruct((_TOTAL, _DSUB), jnp.float32),
    scratch_types=[
        pltpu.VMEM((_NG, _RPG), jnp.int32),
        pltpu.VMEM((4, _BURST, _DSUB), jnp.float32),
        pltpu.SemaphoreType.DMA,
        pltpu.SemaphoreType.DMA,
        pltpu.SemaphoreType.DMA,
        pltpu.SemaphoreType.DMA,
    ],
    compiler_params=pltpu.CompilerParams(use_tc_tiling_on_sc=False),
)
def _pq_gather(codes_hbm, table_hbm, out_hbm, idx_v, rows_v, sem_ga, sem_gb,
               sem_gc, sem_s):
    wid = lax.axis_index("s") * _NC + lax.axis_index("c")

    pltpu.sync_copy(codes_hbm.at[pl.ds(wid * _NG, _NG)], idx_v)

    lane = lax.iota(jnp.int32, 16)

    def add_offsets_burst(gb):
        def row(jj, carry):
            j = gb * _KF + jj
            for o in range(_RPG // 16):
                pos = j * _RPG + (o * 16) + lane
                off = lax.rem(pos, _M) * _K
                sl = pl.ds(o * 16, 16)
                idx_v[j, sl] = idx_v[j, sl] + off
            return carry

        lax.fori_loop(0, _KF, row, 0)

    def fire(g, buf, sem):
        for f in range(_KF):
            pltpu.async_copy(
                table_hbm.at[idx_v.at[g * _KF + f]],
                rows_v.at[buf, pl.ds(f * _RPG, _RPG)],
                sem,
            )

    def drain_gathers(buf, sem):
        pltpu.make_async_copy(
            out_hbm.at[pl.ds(0, _BURST)], rows_v.at[buf], sem
        ).wait()

    def scatter(g, buf):
        pltpu.async_copy(
            rows_v.at[buf],
            out_hbm.at[pl.ds(wid * _PER_W + g * _BURST, _BURST)],
            sem_s,
        )

    def drain_scatter(buf):
        pltpu.make_async_copy(
            rows_v.at[buf], out_hbm.at[pl.ds(wid * _PER_W, _BURST)], sem_s
        ).wait()

    sems = (sem_ga, sem_gb, sem_gc)
    add_offsets_burst(0)
    add_offsets_burst(1)

    def burst_triple(i, carry):
        for b3 in range(3):
            g = 3 * i + b3
            buf = lax.rem(g, 4)

            @pl.when(g >= 4)
            def _scatter_done():
                drain_scatter(buf)

            fire(g, buf, sems[b3])

            @pl.when(g + 2 < _NB)
            def _prep_next():
                add_offsets_burst(g + 2)

            @pl.when(g >= 1)
            def _prev_done():
                prev = lax.rem(g + 3, 4)
                drain_gathers(prev, sems[(b3 + 2) % 3])
                scatter(g - 1, prev)

        return carry

    lax.fori_loop(0, _NB // 3, burst_triple, 0)

    last = lax.rem(_NB - 1, 4)
    drain_gathers(last, sems[(_NB - 1) % 3])
    scatter(_NB - 1, last)
    for _ in range(4):
        drain_scatter(0)


def kernel(doc_codes, sub_weights):
    codes = doc_codes.astype(jnp.int32).reshape(_NW * _NG, _RPG)
    table = sub_weights.reshape(_M * _K, _DSUB)
    flat = _pq_gather(codes, table)
    return flat.reshape(_B, _D)
